# Initial kernel scaffold; baseline (speedup 1.0000x reference)
import jax
import jax.numpy as jnp
from jax.experimental import pallas as pl


def kernel(logits, temperatures, top_ps, min_ps, top_ks, noise):
    raise NotImplementedError("write your pallas kernel here")


# speedup vs baseline: 33.4007x; 33.4007x over previous
import jax
import jax.numpy as jnp
from jax.experimental import pallas as pl
from jax.experimental.pallas import tpu as pltpu

_B = 64
_V = 100000
_ROWS = 8
_LO_SENTINEL = -2139095042
_HI_SENTINEL = 2139095041


def _sampler_body(logits_ref, noise_ref, temp_ref, topp_ref, minp_ref,
                  topk_ref, probs_ref, tok_ref, slp_ref):
    x = logits_ref[...] / temp_ref[...]
    m = jnp.max(x, axis=-1, keepdims=True)
    e = jnp.exp(x - m)
    z = jnp.sum(e, axis=-1, keepdims=True)

    b = jax.lax.bitcast_convert_type(x, jnp.int32)
    u = jnp.where(b >= 0, b, b ^ jnp.int32(0x7FFFFFFF))

    k = topk_ref[...]
    mass_limit = topp_ref[...] * z
    r1 = (_ROWS, 1)
    lo_k = jnp.full(r1, _LO_SENTINEL, jnp.int32)
    hi_k = jnp.full(r1, _HI_SENTINEL, jnp.int32)
    lo_p = jnp.full(r1, _LO_SENTINEL, jnp.int32)
    hi_p = jnp.full(r1, _HI_SENTINEL, jnp.int32)

    def body(_, carry):
        lo_k, hi_k, lo_p, hi_p = carry
        mid_k = (lo_k & hi_k) + ((lo_k ^ hi_k) >> 1)
        mid_p = (lo_p & hi_p) + ((lo_p ^ hi_p) >> 1)
        cnt = jnp.sum((u >= mid_k).astype(jnp.int32), axis=-1, keepdims=True)
        mass = jnp.sum(jnp.where(u > mid_p, e, 0.0), axis=-1, keepdims=True)
        ck = cnt >= k
        lo_k = jnp.where(ck, mid_k, lo_k)
        hi_k = jnp.where(ck, hi_k, mid_k)
        cp = mass <= mass_limit
        hi_p = jnp.where(cp, mid_p, hi_p)
        lo_p = jnp.where(cp, lo_p, mid_p)
        return lo_k, hi_k, lo_p, hi_p

    lo_k, hi_k, lo_p, hi_p = jax.lax.fori_loop(
        0, 32, body, (lo_k, hi_k, lo_p, hi_p))

    keep = (u >= jnp.maximum(lo_k, hi_p)) & (e >= minp_ref[...])
    ez = jnp.where(keep, e, 0.0)
    z2 = jnp.sum(ez, axis=-1, keepdims=True)
    probs = ez * (1.0 / z2)
    probs_ref[...] = probs

    r = probs / (-jnp.log(noise_ref[...]))
    rmax = jnp.max(r, axis=-1, keepdims=True)
    iota = jax.lax.broadcasted_iota(jnp.int32, r.shape, 1)
    idx = jnp.min(jnp.where(r == rmax, iota, _V), axis=-1, keepdims=True)
    tok_ref[...] = idx

    xs = jnp.max(jnp.where(iota == idx, x, -jnp.inf), axis=-1, keepdims=True)
    slp_ref[...] = (xs - m) - jnp.log(z2)


def kernel(logits, temperatures, top_ps, min_ps, top_ks, noise):
    nb = _B // _ROWS
    row_spec = pl.BlockSpec((_ROWS, _V), lambda i: (i, 0))
    par_spec = pl.BlockSpec((_ROWS, 1), lambda i: (i, 0))
    probs, tok, slp = pl.pallas_call(
        _sampler_body,
        grid=(nb,),
        in_specs=[row_spec, row_spec, par_spec, par_spec, par_spec, par_spec],
        out_specs=[row_spec, par_spec, par_spec],
        out_shape=[
            jax.ShapeDtypeStruct((_B, _V), jnp.float32),
            jax.ShapeDtypeStruct((_B, 1), jnp.int32),
            jax.ShapeDtypeStruct((_B, 1), jnp.float32),
        ],
        compiler_params=pltpu.CompilerParams(
            dimension_semantics=("parallel",)),
    )(logits, noise, temperatures.reshape(_B, 1), top_ps.reshape(_B, 1),
      min_ps.reshape(_B, 1), top_ks.reshape(_B, 1))
    return probs, tok.reshape(_B), slp
